# Initial kernel scaffold; baseline (speedup 1.0000x reference)
import jax
import jax.numpy as jnp
from jax.experimental import pallas as pl


def kernel(x, hash_weights):
    raise NotImplementedError("write your pallas kernel here")


# speedup vs baseline: 1.5574x; 1.5574x over previous
import functools

import jax
import jax.numpy as jnp
from jax.experimental import pallas as pl
from jax.experimental.pallas import tpu as pltpu

_NUM_EXPERTS = 8
_NUM_HASHES = 4
_BT = 512


def _body(x_ref, w_ref, probs_ref, logits_ref):
    x = x_ref[...]
    w = w_ref[...]
    hv = jax.lax.dot_general(
        x, w, (((1,), (1,)), ((), ())),
        preferred_element_type=jnp.float32)
    m = jnp.max(hv, axis=-1, keepdims=True)
    iota_h = jax.lax.broadcasted_iota(jnp.int32, hv.shape, 1)
    idx = jnp.min(jnp.where(hv == m, iota_h, _NUM_HASHES), axis=-1,
                  keepdims=True) % _NUM_EXPERTS
    cols = jax.lax.broadcasted_iota(jnp.int32, (x.shape[0], _NUM_EXPERTS), 1)
    onehot = cols == idx
    eps = jnp.float32(1e-8)
    probs_ref[...] = jnp.where(onehot, jnp.float32(1.0), eps)
    logits_ref[...] = jnp.where(onehot, jnp.float32(0.0), jnp.log(eps))


@jax.jit
def kernel(x, hash_weights):
    n, d = x.shape
    grid = (n // _BT,)
    probs, logits = pl.pallas_call(
        _body,
        grid=grid,
        in_specs=[
            pl.BlockSpec((_BT, d), lambda i: (i, 0)),
            pl.BlockSpec((_NUM_HASHES, d), lambda i: (0, 0)),
        ],
        out_specs=[
            pl.BlockSpec((_BT, _NUM_EXPERTS), lambda i: (i, 0)),
            pl.BlockSpec((_BT, _NUM_EXPERTS), lambda i: (i, 0)),
        ],
        out_shape=[
            jax.ShapeDtypeStruct((n, _NUM_EXPERTS), jnp.float32),
            jax.ShapeDtypeStruct((n, _NUM_EXPERTS), jnp.float32),
        ],
        compiler_params=pltpu.CompilerParams(
            dimension_semantics=("parallel",)),
    )(x, hash_weights)
    return (logits, probs)


# speedup vs baseline: 1.7527x; 1.1254x over previous
import functools

import jax
import jax.numpy as jnp
from jax.experimental import pallas as pl
from jax.experimental.pallas import tpu as pltpu

_NUM_EXPERTS = 8
_NUM_HASHES = 4
_BT = 1024


def _body(x_ref, w_ref, probs_ref, logits_ref):
    x = x_ref[...]
    w = w_ref[...]
    hv = jax.lax.dot_general(
        x, w, (((1,), (1,)), ((), ())),
        preferred_element_type=jnp.float32)
    m = jnp.max(hv, axis=-1, keepdims=True)
    iota_h = jax.lax.broadcasted_iota(jnp.int32, hv.shape, 1)
    idx = jnp.min(jnp.where(hv == m, iota_h, _NUM_HASHES), axis=-1,
                  keepdims=True) % _NUM_EXPERTS
    cols = jax.lax.broadcasted_iota(jnp.int32, (x.shape[0], _NUM_EXPERTS), 1)
    onehot = cols == idx
    eps = jnp.float32(1e-8)
    probs_ref[...] = jnp.where(onehot, jnp.float32(1.0), eps)
    logits_ref[...] = jnp.where(onehot, jnp.float32(0.0), jnp.log(eps))


@jax.jit
def kernel(x, hash_weights):
    n, d = x.shape
    grid = (n // _BT,)
    probs, logits = pl.pallas_call(
        _body,
        grid=grid,
        in_specs=[
            pl.BlockSpec((_BT, d), lambda i: (i, 0)),
            pl.BlockSpec((_NUM_HASHES, d), lambda i: (0, 0)),
        ],
        out_specs=[
            pl.BlockSpec((_BT, _NUM_EXPERTS), lambda i: (i, 0)),
            pl.BlockSpec((_BT, _NUM_EXPERTS), lambda i: (i, 0)),
        ],
        out_shape=[
            jax.ShapeDtypeStruct((n, _NUM_EXPERTS), jnp.float32),
            jax.ShapeDtypeStruct((n, _NUM_EXPERTS), jnp.float32),
        ],
        compiler_params=pltpu.CompilerParams(
            dimension_semantics=("parallel",)),
    )(x, hash_weights)
    return (logits, probs)
